# baseline (speedup 1.0000x reference)
import functools

import jax
import jax.numpy as jnp
from jax import lax
from jax.experimental import pallas as pl
from jax.experimental.pallas import tpu as pltpu
from jax.experimental.pallas import tpu_sc as plsc

NUM_FACTOR = 64
BATCH = 16384

_NC = 2
_NS = 16
_NW = _NC * _NS
_ROWS_PER_W = BATCH // _NW
_IDX_CHUNK = 128
_N_CHUNKS = _ROWS_PER_W // _IDX_CHUNK
_L = 16


_GATHER_DNUMS = lax.GatherDimensionNumbers(
    offset_dims=(), collapsed_slice_dims=(0,), start_index_map=(0,))


def _rot_gather(v, idx):
    return lax.gather(v, idx[:, None], _GATHER_DNUMS, slice_sizes=(1,),
                      mode=lax.GatherScatterMode.PROMISE_IN_BOUNDS)


def _gmf_body(uidx_hbm, iidx_hbm, utab_hbm, itab_hbm, w_hbm, out_hbm,
              uidx_v, iidx_v, urows_v, irows_v, w_v, out_v, sem):
    wid = lax.axis_index("s") * _NC + lax.axis_index("c")

    pltpu.sync_copy(uidx_hbm.at[wid], uidx_v)
    pltpu.sync_copy(iidx_hbm.at[wid], iidx_v)
    pltpu.sync_copy(w_hbm, w_v)

    copies = []
    for k in range(_N_CHUNKS):
        copies.append(pltpu.async_copy(
            utab_hbm.at[uidx_v.at[k]],
            urows_v.at[pl.ds(k * _IDX_CHUNK, _IDX_CHUNK)], sem))
        copies.append(pltpu.async_copy(
            itab_hbm.at[iidx_v.at[k]],
            irows_v.at[pl.ds(k * _IDX_CHUNK, _IDX_CHUNK)], sem))
    for c in copies:
        c.wait()

    w0 = w_v[pl.ds(0, _L)]
    w1 = w_v[pl.ds(_L, _L)]
    w2 = w_v[pl.ds(2 * _L, _L)]
    w3 = w_v[pl.ds(3 * _L, _L)]

    lane_ids = lax.iota(jnp.int32, _L)
    onehot = [lane_ids == l for l in range(_L)]
    rot_idx = [(lane_ids + sh) & (_L - 1) for sh in (8, 4, 2, 1)]

    def block(b, carry):
        base = b * _L
        acc = jnp.zeros((_L,), jnp.float32)
        for l in range(_L):
            r = base + l
            v = (urows_v[r, pl.ds(0, _L)] * irows_v[r, pl.ds(0, _L)] * w0
                 + urows_v[r, pl.ds(_L, _L)] * irows_v[r, pl.ds(_L, _L)] * w1
                 + urows_v[r, pl.ds(2 * _L, _L)] * irows_v[r, pl.ds(2 * _L, _L)] * w2
                 + urows_v[r, pl.ds(3 * _L, _L)] * irows_v[r, pl.ds(3 * _L, _L)] * w3)
            for idx in rot_idx:
                v = v + _rot_gather(v, idx)
            acc = jnp.where(onehot[l], v, acc)
        out_v[pl.ds(base, _L)] = acc
        return carry

    lax.fori_loop(0, _ROWS_PER_W // _L, block, 0)

    pltpu.sync_copy(out_v, out_hbm.at[pl.ds(wid * _ROWS_PER_W, _ROWS_PER_W)])


@jax.jit
def _gmf(user, item, user_table, item_table, w_flat):
    uidx = user.reshape(_NW, _N_CHUNKS, _IDX_CHUNK)
    iidx = item.reshape(_NW, _N_CHUNKS, _IDX_CHUNK)
    mesh = plsc.VectorSubcoreMesh(core_axis_name="c", subcore_axis_name="s")
    run = functools.partial(
        pl.kernel, mesh=mesh,
        compiler_params=pltpu.CompilerParams(use_tc_tiling_on_sc=False),
        out_type=jax.ShapeDtypeStruct((BATCH,), jnp.float32),
        scratch_types=[
            pltpu.VMEM((_N_CHUNKS, _IDX_CHUNK), jnp.int32),
            pltpu.VMEM((_N_CHUNKS, _IDX_CHUNK), jnp.int32),
            pltpu.VMEM((_ROWS_PER_W, NUM_FACTOR), jnp.float32),
            pltpu.VMEM((_ROWS_PER_W, NUM_FACTOR), jnp.float32),
            pltpu.VMEM((NUM_FACTOR,), jnp.float32),
            pltpu.VMEM((_ROWS_PER_W,), jnp.float32),
            pltpu.SemaphoreType.DMA,
        ],
    )(_gmf_body)
    return run(uidx, iidx, user_table, item_table, w_flat)


def kernel(user, item, user_table, item_table, W):
    return _gmf(user, item, user_table, item_table, W.reshape(-1))
